# baseline (speedup 1.0000x reference)
import functools

import jax
import jax.numpy as jnp
from jax import lax
from jax.experimental import pallas as pl
from jax.experimental.pallas import tpu as pltpu
from jax.experimental.pallas import tpu_sc as plsc

_N_ADDR = 65536
_D = 1024
_TEMPERATURE = 0.1
_EPS = 1e-8

_N_TC = 45056
_N_SC = _N_ADDR - _N_TC

_BLK = 4096
_TC_GRID = _N_TC // _BLK

_SC_TILES = 32
_C = 32
_R_TILE = _N_SC // _SC_TILES


def _tc_body(q_ref, a_ref, acc_out, den_out, acc_ref, den_ref):
    i = pl.program_id(0)

    @pl.when(i == 0)
    def _init():
        acc_ref[...] = jnp.zeros_like(acc_ref)
        den_ref[...] = jnp.zeros_like(den_ref)

    a = a_ref[...]
    q = q_ref[...]
    q_norm = jnp.maximum(jnp.sqrt(jnp.sum(q * q)), _EPS)
    s = lax.dot_general(
        a, q, (((1,), (1,)), ((), ())),
        preferred_element_type=jnp.float32,
    )
    n2 = lax.dot_general(
        a * a, jnp.ones_like(q), (((1,), (1,)), ((), ())),
        preferred_element_type=jnp.float32,
    )
    a_norm = jnp.maximum(jnp.sqrt(n2), _EPS)
    cos = s / (a_norm * q_norm)
    w = jnp.exp((cos - 1.0) / _TEMPERATURE)
    acc_ref[...] += lax.dot_general(
        w, a, (((0,), (0,)), ((), ())),
        preferred_element_type=jnp.float32,
    )
    den_ref[...] += jnp.sum(w)

    @pl.when(i == _TC_GRID - 1)
    def _fin():
        acc_out[...] = acc_ref[...]
        den_out[...] = den_ref[...]


def _tc_partial(query2d, addresses):
    return pl.pallas_call(
        _tc_body,
        grid=(_TC_GRID,),
        in_specs=[
            pl.BlockSpec((1, _D), lambda i: (0, 0)),
            pl.BlockSpec((_BLK, _D), lambda i: (i, 0)),
        ],
        out_specs=[
            pl.BlockSpec((1, _D), lambda i: (0, 0)),
            pl.BlockSpec((1, 1), lambda i: (0, 0)),
        ],
        out_shape=[
            jax.ShapeDtypeStruct((1, _D), jnp.float32),
            jax.ShapeDtypeStruct((1, 1), jnp.float32),
        ],
        scratch_shapes=[
            pltpu.VMEM((1, _D), jnp.float32),
            pltpu.VMEM((1, 1), jnp.float32),
        ],
        compiler_params=pltpu.CompilerParams(
            dimension_semantics=("arbitrary",),
        ),
    )(query2d, addresses)


def _rsqrt16(x):
    i = plsc.bitcast(x, jnp.int32)
    i = jnp.int32(0x5F3759DF) - (i >> 1)
    y = plsc.bitcast(i, jnp.float32)
    for _ in range(3):
        y = y * (1.5 - 0.5 * x * y * y)
    return y


def _sc_partial(query, addresses):
    mesh = plsc.VectorSubcoreMesh(core_axis_name="c", subcore_axis_name="s")

    @functools.partial(
        pl.kernel,
        out_type=[
            jax.ShapeDtypeStruct((_SC_TILES, _D), jnp.float32),
            jax.ShapeDtypeStruct((_SC_TILES, 16), jnp.float32),
        ],
        mesh=mesh,
        scratch_types=[
            pltpu.VMEM((_D,), jnp.float32),
            pltpu.VMEM((_C, _D), jnp.float32),
            pltpu.VMEM((_C, _D), jnp.float32),
            pltpu.VMEM((_D,), jnp.float32),
            pltpu.VMEM((16,), jnp.float32),
            pltpu.VMEM((256,), jnp.float32),
            pltpu.VMEM((256,), jnp.float32),
            pltpu.SemaphoreType.DMA,
            pltpu.SemaphoreType.DMA,
        ],
        compiler_params=pltpu.CompilerParams(needs_layout_passes=False),
    )
    def sc_kernel(q_hbm, a_hbm, accs_hbm, dens_hbm,
                  q_v, a0, a1, acc_v, den_v, t_s, t_n, sem0, sem1):
        wid = lax.axis_index("s") * 2 + lax.axis_index("c")
        base = _N_TC + wid * _R_TILE

        pltpu.sync_copy(q_hbm, q_v)

        zero = jnp.zeros((16,), jnp.float32)
        for k in range(_D // 16):
            acc_v[pl.ds(16 * k, 16)] = zero

        def _qn_body(k, acc):
            qk = q_v[pl.ds(16 * k, 16)]
            return acc + qk * qk
        qn2 = jnp.sum(lax.fori_loop(0, _D // 16, _qn_body, zero))
        qrn = jnp.minimum(
            _rsqrt16(jnp.maximum(jnp.full((16,), qn2), 1e-30)), 1.0 / _EPS)

        lanes = lax.iota(jnp.int32, 16)

        def _process(a_buf, den):
            for h in range(_C // 16):
                r0 = 16 * h
                for g in range(2):
                    rg = r0 + 8 * g

                    def _pa(k, carry, _rg=rg):
                        sl = pl.ds(16 * k, 16)
                        qk = q_v[sl]
                        out = []
                        for r in range(8):
                            ar = a_buf[_rg + r, sl]
                            out.append(carry[2 * r] + ar * qk)
                            out.append(carry[2 * r + 1] + ar * ar)
                        return tuple(out)

                    accs = lax.fori_loop(0, _D // 16, _pa, (zero,) * 16)
                    for r in range(8):
                        t_s[pl.ds(16 * (8 * g + r), 16)] = accs[2 * r]
                        t_n[pl.ds(16 * (8 * g + r), 16)] = accs[2 * r + 1]
                s_vec = zero
                n_vec = zero
                for j in range(16):
                    idx = lanes * 16 + j
                    s_vec = s_vec + plsc.load_gather(t_s, [idx])
                    n_vec = n_vec + plsc.load_gather(t_n, [idx])
                rn = jnp.minimum(
                    _rsqrt16(jnp.maximum(n_vec, 1e-30)), 1.0 / _EPS)
                cos = s_vec * rn * qrn
                w_vec = jnp.exp((cos - 1.0) * (1.0 / _TEMPERATURE))
                den = den + w_vec
                wb = [jnp.full((16,), w_vec[r]) for r in range(16)]

                def _pb(k, c, _r0=r0, _wb=wb):
                    sl = pl.ds(16 * k, 16)
                    acc = acc_v[sl]
                    for r in range(16):
                        acc = acc + _wb[r] * a_buf[_r0 + r, sl]
                    acc_v[sl] = acc
                    return c

                lax.fori_loop(0, _D // 16, _pb, 0)
            return den

        nch = _R_TILE // _C
        pltpu.async_copy(a_hbm.at[pl.ds(base, _C)], a0, sem0)

        def _chunk_pair(i, den):
            c0 = 2 * i
            pltpu.async_copy(
                a_hbm.at[pl.ds(base + (c0 + 1) * _C, _C)], a1, sem1)
            pltpu.make_async_copy(
                a_hbm.at[pl.ds(base, _C)], a0, sem0).wait()
            den = _process(a0, den)

            @pl.when(c0 + 2 < nch)
            def _():
                pltpu.async_copy(
                    a_hbm.at[pl.ds(base + (c0 + 2) * _C, _C)], a0, sem0)
            pltpu.make_async_copy(
                a_hbm.at[pl.ds(base, _C)], a1, sem1).wait()
            return _process(a1, den)

        den_acc = lax.fori_loop(0, nch // 2, _chunk_pair, zero)

        pltpu.sync_copy(acc_v, accs_hbm.at[wid])
        den_v[...] = den_acc
        pltpu.sync_copy(den_v, dens_hbm.at[wid])

    return sc_kernel(query, addresses)


def _merge_body(acc_tc_ref, den_tc_ref, accs_sc_ref, dens_sc_ref, o_ref):
    acc = acc_tc_ref[...] + jnp.sum(accs_sc_ref[...], axis=0, keepdims=True)
    den = den_tc_ref[0, 0] + jnp.sum(dens_sc_ref[...])
    o_ref[...] = acc / den


@jax.jit
def kernel(query_address, addresses):
    q2d = query_address.reshape(1, _D)
    acc_tc, den_tc = _tc_partial(q2d, addresses)
    accs_sc, dens_sc = _sc_partial(query_address, addresses)
    out = pl.pallas_call(
        _merge_body,
        out_shape=jax.ShapeDtypeStruct((1, _D), jnp.float32),
    )(acc_tc, den_tc, accs_sc, dens_sc)
    return out.reshape(_D)


# speedup vs baseline: 1.2201x; 1.2201x over previous
import functools

import jax
import jax.numpy as jnp
from jax import lax
from jax.experimental import pallas as pl
from jax.experimental.pallas import tpu as pltpu
from jax.experimental.pallas import tpu_sc as plsc

_N_ADDR = 65536
_D = 1024
_TEMPERATURE = 0.1
_EPS = 1e-8

_N_TC = 53248
_N_SC = _N_ADDR - _N_TC

_BLK = 4096
_TC_GRID = _N_TC // _BLK

_SC_TILES = 32
_C = 32
_R_TILE = _N_SC // _SC_TILES


def _tc_body(q_ref, a_ref, acc_out, den_out, acc_ref, den_ref):
    i = pl.program_id(0)

    @pl.when(i == 0)
    def _init():
        acc_ref[...] = jnp.zeros_like(acc_ref)
        den_ref[...] = jnp.zeros_like(den_ref)

    a = a_ref[...]
    q = q_ref[...]
    q_norm = jnp.maximum(jnp.sqrt(jnp.sum(q * q)), _EPS)
    s = lax.dot_general(
        a, q, (((1,), (1,)), ((), ())),
        preferred_element_type=jnp.float32,
    )
    n2 = lax.dot_general(
        a * a, jnp.ones_like(q), (((1,), (1,)), ((), ())),
        preferred_element_type=jnp.float32,
    )
    a_norm = jnp.maximum(jnp.sqrt(n2), _EPS)
    cos = s / (a_norm * q_norm)
    w = jnp.exp((cos - 1.0) / _TEMPERATURE)
    acc_ref[...] += lax.dot_general(
        w, a, (((0,), (0,)), ((), ())),
        preferred_element_type=jnp.float32,
    )
    den_ref[...] += jnp.sum(w)

    @pl.when(i == _TC_GRID - 1)
    def _fin():
        acc_out[...] = acc_ref[...]
        den_out[...] = den_ref[...]


def _tc_partial(query2d, addresses):
    return pl.pallas_call(
        _tc_body,
        grid=(_TC_GRID,),
        in_specs=[
            pl.BlockSpec((1, _D), lambda i: (0, 0)),
            pl.BlockSpec((_BLK, _D), lambda i: (i, 0)),
        ],
        out_specs=[
            pl.BlockSpec((1, _D), lambda i: (0, 0)),
            pl.BlockSpec((1, 1), lambda i: (0, 0)),
        ],
        out_shape=[
            jax.ShapeDtypeStruct((1, _D), jnp.float32),
            jax.ShapeDtypeStruct((1, 1), jnp.float32),
        ],
        scratch_shapes=[
            pltpu.VMEM((1, _D), jnp.float32),
            pltpu.VMEM((1, 1), jnp.float32),
        ],
        compiler_params=pltpu.CompilerParams(
            dimension_semantics=("arbitrary",),
        ),
    )(query2d, addresses)


def _rsqrt16(x):
    i = plsc.bitcast(x, jnp.int32)
    i = jnp.int32(0x5F3759DF) - (i >> 1)
    y = plsc.bitcast(i, jnp.float32)
    for _ in range(3):
        y = y * (1.5 - 0.5 * x * y * y)
    return y


def _sc_partial(query, addresses):
    mesh = plsc.VectorSubcoreMesh(core_axis_name="c", subcore_axis_name="s")

    @functools.partial(
        pl.kernel,
        out_type=[
            jax.ShapeDtypeStruct((_SC_TILES, _D), jnp.float32),
            jax.ShapeDtypeStruct((_SC_TILES, 16), jnp.float32),
        ],
        mesh=mesh,
        scratch_types=[
            pltpu.VMEM((_D,), jnp.float32),
            pltpu.VMEM((_C, _D), jnp.float32),
            pltpu.VMEM((_C, _D), jnp.float32),
            pltpu.VMEM((_D,), jnp.float32),
            pltpu.VMEM((16,), jnp.float32),
            pltpu.VMEM((256,), jnp.float32),
            pltpu.VMEM((256,), jnp.float32),
            pltpu.SemaphoreType.DMA,
            pltpu.SemaphoreType.DMA,
        ],
        compiler_params=pltpu.CompilerParams(needs_layout_passes=False),
    )
    def sc_kernel(q_hbm, a_hbm, accs_hbm, dens_hbm,
                  q_v, a0, a1, acc_v, den_v, t_s, t_n, sem0, sem1):
        wid = lax.axis_index("s") * 2 + lax.axis_index("c")
        base = _N_TC + wid * _R_TILE

        pltpu.sync_copy(q_hbm, q_v)

        zero = jnp.zeros((16,), jnp.float32)
        for k in range(_D // 16):
            acc_v[pl.ds(16 * k, 16)] = zero

        def _qn_body(k, acc):
            qk = q_v[pl.ds(16 * k, 16)]
            return acc + qk * qk
        qn2 = jnp.sum(lax.fori_loop(0, _D // 16, _qn_body, zero))
        qrn = jnp.minimum(
            _rsqrt16(jnp.maximum(jnp.full((16,), qn2), 1e-30)), 1.0 / _EPS)

        lanes = lax.iota(jnp.int32, 16)

        def _process(a_buf, den):
            for h in range(_C // 16):
                r0 = 16 * h
                for g in range(2):
                    rg = r0 + 8 * g

                    def _pa(k, carry, _rg=rg):
                        sl = pl.ds(16 * k, 16)
                        qk = q_v[sl]
                        out = []
                        for r in range(8):
                            ar = a_buf[_rg + r, sl]
                            out.append(carry[2 * r] + ar * qk)
                            out.append(carry[2 * r + 1] + ar * ar)
                        return tuple(out)

                    accs = lax.fori_loop(0, _D // 16, _pa, (zero,) * 16)
                    for r in range(8):
                        t_s[pl.ds(16 * (8 * g + r), 16)] = accs[2 * r]
                        t_n[pl.ds(16 * (8 * g + r), 16)] = accs[2 * r + 1]
                s_vec = zero
                n_vec = zero
                for j in range(16):
                    idx = lanes * 16 + j
                    s_vec = s_vec + plsc.load_gather(t_s, [idx])
                    n_vec = n_vec + plsc.load_gather(t_n, [idx])
                rn = jnp.minimum(
                    _rsqrt16(jnp.maximum(n_vec, 1e-30)), 1.0 / _EPS)
                cos = s_vec * rn * qrn
                w_vec = jnp.exp((cos - 1.0) * (1.0 / _TEMPERATURE))
                den = den + w_vec
                wb = [jnp.full((16,), w_vec[r]) for r in range(16)]

                def _pb(k, c, _r0=r0, _wb=wb):
                    sl = pl.ds(16 * k, 16)
                    acc = acc_v[sl]
                    for r in range(16):
                        acc = acc + _wb[r] * a_buf[_r0 + r, sl]
                    acc_v[sl] = acc
                    return c

                lax.fori_loop(0, _D // 16, _pb, 0)
            return den

        nch = _R_TILE // _C
        pltpu.async_copy(a_hbm.at[pl.ds(base, _C)], a0, sem0)

        def _chunk_pair(i, den):
            c0 = 2 * i
            pltpu.async_copy(
                a_hbm.at[pl.ds(base + (c0 + 1) * _C, _C)], a1, sem1)
            pltpu.make_async_copy(
                a_hbm.at[pl.ds(base, _C)], a0, sem0).wait()
            den = _process(a0, den)

            @pl.when(c0 + 2 < nch)
            def _():
                pltpu.async_copy(
                    a_hbm.at[pl.ds(base + (c0 + 2) * _C, _C)], a0, sem0)
            pltpu.make_async_copy(
                a_hbm.at[pl.ds(base, _C)], a1, sem1).wait()
            return _process(a1, den)

        den_acc = lax.fori_loop(0, nch // 2, _chunk_pair, zero)

        pltpu.sync_copy(acc_v, accs_hbm.at[wid])
        den_v[...] = den_acc
        pltpu.sync_copy(den_v, dens_hbm.at[wid])

    return sc_kernel(query, addresses)


def _merge_body(acc_tc_ref, den_tc_ref, accs_sc_ref, dens_sc_ref, o_ref):
    acc = acc_tc_ref[...] + jnp.sum(accs_sc_ref[...], axis=0, keepdims=True)
    den = den_tc_ref[0, 0] + jnp.sum(dens_sc_ref[...])
    o_ref[...] = acc / den


@jax.jit
def kernel(query_address, addresses):
    q2d = query_address.reshape(1, _D)
    acc_tc, den_tc = _tc_partial(q2d, addresses)
    accs_sc, dens_sc = _sc_partial(query_address, addresses)
    out = pl.pallas_call(
        _merge_body,
        out_shape=jax.ShapeDtypeStruct((1, _D), jnp.float32),
    )(acc_tc, den_tc, accs_sc, dens_sc)
    return out.reshape(_D)
